# baseline (speedup 1.0000x reference)
import functools

import jax
import jax.numpy as jnp
import numpy as np
from jax.experimental import pallas as pl
from jax.experimental.pallas import tpu as pltpu
from jax.experimental.pallas import tpu_sc as plsc
from jax.sharding import Mesh, PartitionSpec as P

_B, _L, _C = 32, 450, 768
_MID = 256
_K = 8912
_KP = 9216
_N = _B * _L
_TN = 1800
_TK = 1024
_NK = _KP // _TK
_CC = 1024
_INV_TEMP = -1.0 / 0.07
_EPS = 1e-12

_INTERPRET = False
_PREC = jax.lax.Precision.DEFAULT


def _flash_body(zn_ref, en_ref, dmin_ref, idx_ref, se_ref,
                rmin_ref, ridx_ref, rsum_ref):
    k = pl.program_id(1)

    @pl.when(k == 0)
    def _():
        rmin_ref[...] = jnp.full((_TN, 1), 3e38, jnp.float32)
        ridx_ref[...] = jnp.full((_TN, 1), 2**30, jnp.int32)
        rsum_ref[...] = jnp.zeros((_TN, 1), jnp.float32)

    en_k = en_ref[:, pl.ds(k * _TK, _TK)]
    s = jnp.dot(zn_ref[...], en_k, precision=_PREC,
                preferred_element_type=jnp.float32)
    d = -2.0 * s + 2.0
    tmin = jnp.min(d, axis=1, keepdims=True)
    lane = jax.lax.broadcasted_iota(jnp.int32, (1, _TK), 1)
    tidx = jnp.min(jnp.where(d == tmin, lane, jnp.int32(2**30)),
                   axis=1, keepdims=True) + k * _TK
    e = jnp.exp(d * jnp.float32(_INV_TEMP))
    tsum = jnp.sum(e, axis=1, keepdims=True)

    better = tmin < rmin_ref[...]
    ridx_ref[...] = jnp.where(better, tidx, ridx_ref[...])
    rmin_ref[...] = jnp.where(better, tmin, rmin_ref[...])
    rsum_ref[...] = rsum_ref[...] + tsum

    @pl.when(k == _NK - 1)
    def _():
        dmin_ref[...] = rmin_ref[...]
        idx_ref[...] = ridx_ref[...]
        se_ref[...] = rsum_ref[...]


def _scalars_body(dmin_ref, se_ref, sdm_ref, st1_ref):
    dm = dmin_ref[...]
    padc = jnp.float32(_KP - _K) * jnp.exp(
        jnp.float32(2.0) * jnp.float32(_INV_TEMP))
    se = se_ref[...] - padc
    sdm_ref[...] = jnp.sum(dm).reshape(1, 1)
    t1 = jnp.exp(dm * jnp.float32(_INV_TEMP)) / se
    st1_ref[...] = jnp.sum(t1).reshape(1, 1)


def _presence_body(idx_ref, pres_ref, *, nt):
    j = pl.program_id(0)
    codes = jax.lax.broadcasted_iota(jnp.int32, (1, _CC), 1) + j * _CC

    def tok_tile(i, acc):
        chunk = idx_ref[pl.ds(i * _TN, _TN), :]
        eq = (chunk == codes).astype(jnp.float32)
        return jnp.maximum(acc, jnp.max(eq, axis=0, keepdims=True))

    pres = jax.lax.fori_loop(
        0, nt, tok_tile, jnp.zeros((1, _CC), jnp.float32))
    pres_ref[...] = pres.reshape(1, 1, _CC)


def _up_body(zq_ref, wup_ref, bup_ref, out_ref):
    out_ref[...] = jnp.dot(zq_ref[...], wup_ref[...], precision=_PREC,
                           preferred_element_type=jnp.float32) + bup_ref[...]


def _sc_gather(enP, idx2, ng):
    mesh = plsc.VectorSubcoreMesh(core_axis_name="core",
                                  subcore_axis_name="subcore")
    win = 128

    @functools.partial(
        pl.kernel,
        out_type=jax.ShapeDtypeStruct((ng, _MID), jnp.float32),
        mesh=mesh)
    def gk(en_hbm, i_hbm, o_hbm):
        def body(i_vmem, o_vmem):
            pltpu.sync_copy(en_hbm.at[i_vmem.at[0]], o_vmem)

        pltpu.emit_pipeline(
            body,
            grid=(ng // win,),
            in_specs=[pl.BlockSpec((1, win), lambda i: (0, i))],
            out_specs=[pl.BlockSpec((win, _MID), lambda i: (i, 0))],
            core_axis_name=("core", "subcore"),
            dimension_semantics=(pltpu.PARALLEL,),
        )(i_hbm, o_hbm)

    return gk(enP, idx2)


def _shard_pipeline(xs, emb, W_down, b_down, W_up, b_up):
    b_loc = xs.shape[0]
    nloc = b_loc * _L
    nt = nloc // _TN

    z = xs @ W_down + b_down
    zf = z.reshape(nloc, _MID)
    zn = zf / jnp.maximum(
        jnp.sqrt(jnp.sum(zf * zf, axis=-1, keepdims=True)), _EPS)
    en = emb / jnp.maximum(
        jnp.sqrt(jnp.sum(emb * emb, axis=-1, keepdims=True)), _EPS)
    enP = jnp.pad(en, ((0, _KP - _K), (0, 0)))
    enT = enP.T

    dmin, idx, se = pl.pallas_call(
        _flash_body,
        grid=(nt, _NK),
        in_specs=[
            pl.BlockSpec((_TN, _MID), lambda t, k: (t, 0)),
            pl.BlockSpec((_MID, _KP), lambda t, k: (0, 0)),
        ],
        out_specs=[
            pl.BlockSpec((_TN, 1), lambda t, k: (t, 0)),
            pl.BlockSpec((_TN, 1), lambda t, k: (t, 0)),
            pl.BlockSpec((_TN, 1), lambda t, k: (t, 0)),
        ],
        out_shape=[
            jax.ShapeDtypeStruct((nloc, 1), jnp.float32),
            jax.ShapeDtypeStruct((nloc, 1), jnp.int32),
            jax.ShapeDtypeStruct((nloc, 1), jnp.float32),
        ],
        scratch_shapes=[
            pltpu.VMEM((_TN, 1), jnp.float32),
            pltpu.VMEM((_TN, 1), jnp.int32),
            pltpu.VMEM((_TN, 1), jnp.float32),
        ],
        interpret=_INTERPRET,
    )(zn, enT)

    sdm, st1 = pl.pallas_call(
        _scalars_body,
        in_specs=[
            pl.BlockSpec((nloc, 1), lambda: (0, 0)),
            pl.BlockSpec((nloc, 1), lambda: (0, 0)),
        ],
        out_specs=[
            pl.BlockSpec((1, 1), lambda: (0, 0)),
            pl.BlockSpec((1, 1), lambda: (0, 0)),
        ],
        out_shape=[
            jax.ShapeDtypeStruct((1, 1), jnp.float32),
            jax.ShapeDtypeStruct((1, 1), jnp.float32),
        ],
        interpret=_INTERPRET,
    )(dmin, se)

    pres = pl.pallas_call(
        functools.partial(_presence_body, nt=nt),
        grid=(_KP // _CC,),
        in_specs=[pl.BlockSpec((nloc, 1), lambda j: (0, 0))],
        out_specs=pl.BlockSpec((1, 1, _CC), lambda j: (j, 0, 0)),
        out_shape=jax.ShapeDtypeStruct((_KP // _CC, 1, _CC), jnp.float32),
        interpret=_INTERPRET,
    )(idx)

    if _INTERPRET:
        zq = jnp.take(enP, idx[:, 0], axis=0)
    else:
        ng = ((nloc + 127) // 128) * 128
        idxp = jnp.pad(idx.reshape(1, nloc), ((0, 0), (0, ng - nloc)))
        zq = _sc_gather(enP, idxp, ng)[:nloc]

    out2d = pl.pallas_call(
        _up_body,
        grid=(nt,),
        in_specs=[
            pl.BlockSpec((_TN, _MID), lambda t: (t, 0)),
            pl.BlockSpec((_MID, _C), lambda t: (0, 0)),
            pl.BlockSpec((1, _C), lambda t: (0, 0)),
        ],
        out_specs=pl.BlockSpec((_TN, _C), lambda t: (t, 0)),
        out_shape=jax.ShapeDtypeStruct((nloc, _C), jnp.float32),
        interpret=_INTERPRET,
    )(zq, W_up, b_up.reshape(1, _C))

    out = out2d.reshape(b_loc, _L, _C)

    sdm = jax.lax.psum(sdm[0, 0], "d")
    st1 = jax.lax.psum(st1[0, 0], "d")
    pres_v = jax.lax.pmax(pres.reshape(1, _KP), "d")

    vq = sdm / jnp.float32(_N * _MID)
    top1 = st1 / jnp.float32(_N)
    usage = jnp.sum(pres_v) / jnp.float32(_K)
    return out, vq, usage, top1


def kernel(x, calib, emb, W_down, b_down, W_up, b_up):
    del calib
    nd = 1
    for cand in (8, 4, 2):
        if len(jax.devices()) >= cand:
            nd = cand
            break
    mesh = Mesh(np.array(jax.devices()[:nd]), ("d",))
    out, vq, usage, top1 = jax.shard_map(
        _shard_pipeline,
        mesh=mesh,
        in_specs=(P("d"), P(), P(), P(), P(), P()),
        out_specs=(P("d"), P(), P(), P()),
        check_vma=False,
    )(x, emb, W_down, b_down, W_up, b_up)
    return (out, vq, 0.25 * vq, jnp.float32(0.0), usage, top1)


# speedup vs baseline: 1.4510x; 1.4510x over previous
import functools

import jax
import jax.numpy as jnp
import numpy as np
from jax.experimental import pallas as pl
from jax.experimental.pallas import tpu as pltpu
from jax.experimental.pallas import tpu_sc as plsc
from jax.sharding import Mesh, PartitionSpec as P

_B, _L, _C = 32, 450, 768
_MID = 256
_K = 8912
_KP = 9216
_N = _B * _L
_TN = 1800
_TK = 1024
_NK = _KP // _TK
_CC = 1024
_INV_TEMP = -1.0 / 0.07
_EPS = 1e-12

_INTERPRET = False
_PREC = jax.lax.Precision.DEFAULT


def _flash_body(zn_ref, en_ref, dmin_ref, idx_ref, se_ref,
                rmin_ref, ridx_ref, rsum_ref):
    k = pl.program_id(1)

    @pl.when(k == 0)
    def _():
        rmin_ref[...] = jnp.full((_TN, 1), 3e38, jnp.float32)
        ridx_ref[...] = jnp.full((_TN, 1), 2**30, jnp.int32)
        rsum_ref[...] = jnp.zeros((_TN, 1), jnp.float32)

    en_k = en_ref[:, pl.ds(k * _TK, _TK)]
    s = jnp.dot(zn_ref[...], en_k, precision=_PREC,
                preferred_element_type=jnp.float32)
    d = -2.0 * s + 2.0
    tmin = jnp.min(d, axis=1, keepdims=True)
    lane = jax.lax.broadcasted_iota(jnp.int32, (1, _TK), 1)
    tidx = jnp.min(jnp.where(d == tmin, lane, jnp.int32(2**30)),
                   axis=1, keepdims=True) + k * _TK
    e = jnp.exp(d * jnp.float32(_INV_TEMP))
    tsum = jnp.sum(e, axis=1, keepdims=True)

    better = tmin < rmin_ref[...]
    ridx_ref[...] = jnp.where(better, tidx, ridx_ref[...])
    rmin_ref[...] = jnp.where(better, tmin, rmin_ref[...])
    rsum_ref[...] = rsum_ref[...] + tsum

    @pl.when(k == _NK - 1)
    def _():
        dmin_ref[...] = rmin_ref[...]
        idx_ref[...] = ridx_ref[...]
        se_ref[...] = rsum_ref[...]


def _scalars_body(dmin_ref, se_ref, sdm_ref, st1_ref):
    dm = dmin_ref[...]
    padc = jnp.float32(_KP - _K) * jnp.exp(
        jnp.float32(2.0) * jnp.float32(_INV_TEMP))
    se = se_ref[...] - padc
    sdm_ref[...] = jnp.sum(dm).reshape(1, 1)
    t1 = jnp.exp(dm * jnp.float32(_INV_TEMP)) / se
    st1_ref[...] = jnp.sum(t1).reshape(1, 1)


def _presence_body(idx_ref, pres_ref, *, nt):
    j = pl.program_id(0)
    codes = jax.lax.broadcasted_iota(jnp.int32, (1, _CC), 1) + j * _CC

    def tok_tile(i, acc):
        chunk = idx_ref[pl.ds(i * _TN, _TN), :]
        eq = (chunk == codes).astype(jnp.float32)
        return jnp.maximum(acc, jnp.max(eq, axis=0, keepdims=True))

    pres = jax.lax.fori_loop(
        0, nt, tok_tile, jnp.zeros((1, _CC), jnp.float32))
    pres_ref[...] = pres.reshape(1, 1, _CC)


def _up_body(zq_ref, wup_ref, bup_ref, out_ref):
    out_ref[...] = jnp.dot(zq_ref[...], wup_ref[...], precision=_PREC,
                           preferred_element_type=jnp.float32) + bup_ref[...]


def _sc_gather(enP, idx2, ng):
    mesh = plsc.VectorSubcoreMesh(core_axis_name="core",
                                  subcore_axis_name="subcore")
    win = 128

    @functools.partial(
        pl.kernel,
        out_type=jax.ShapeDtypeStruct((ng, _MID), jnp.float32),
        mesh=mesh)
    def gk(en_hbm, i_hbm, o_hbm):
        def body(i_vmem, o_vmem):
            pltpu.sync_copy(en_hbm.at[i_vmem.at[0]], o_vmem)

        pltpu.emit_pipeline(
            body,
            grid=(ng // win,),
            in_specs=[pl.BlockSpec((1, win), lambda i: (0, i))],
            out_specs=[pl.BlockSpec((win, _MID), lambda i: (i, 0))],
            core_axis_name=("core", "subcore"),
            dimension_semantics=(pltpu.PARALLEL,),
        )(i_hbm, o_hbm)

    return gk(enP, idx2)


def _shard_pipeline(xs, emb, W_down, b_down, W_up, b_up):
    b_loc = xs.shape[0]
    nloc = b_loc * _L
    nt = nloc // _TN

    z = xs @ W_down + b_down
    zf = z.reshape(nloc, _MID)
    zn = zf / jnp.maximum(
        jnp.sqrt(jnp.sum(zf * zf, axis=-1, keepdims=True)), _EPS)
    en = emb / jnp.maximum(
        jnp.sqrt(jnp.sum(emb * emb, axis=-1, keepdims=True)), _EPS)
    enP = jnp.pad(en, ((0, _KP - _K), (0, 0)))
    enT = enP.T

    dmin, idx, se = pl.pallas_call(
        _flash_body,
        grid=(nt, _NK),
        in_specs=[
            pl.BlockSpec((_TN, _MID), lambda t, k: (t, 0)),
            pl.BlockSpec((_MID, _KP), lambda t, k: (0, 0)),
        ],
        out_specs=[
            pl.BlockSpec((_TN, 1), lambda t, k: (t, 0)),
            pl.BlockSpec((_TN, 1), lambda t, k: (t, 0)),
            pl.BlockSpec((_TN, 1), lambda t, k: (t, 0)),
        ],
        out_shape=[
            jax.ShapeDtypeStruct((nloc, 1), jnp.float32),
            jax.ShapeDtypeStruct((nloc, 1), jnp.int32),
            jax.ShapeDtypeStruct((nloc, 1), jnp.float32),
        ],
        scratch_shapes=[
            pltpu.VMEM((_TN, 1), jnp.float32),
            pltpu.VMEM((_TN, 1), jnp.int32),
            pltpu.VMEM((_TN, 1), jnp.float32),
        ],
        interpret=_INTERPRET,
    )(zn, enT)

    sdm, st1 = pl.pallas_call(
        _scalars_body,
        in_specs=[
            pl.BlockSpec((nloc, 1), lambda: (0, 0)),
            pl.BlockSpec((nloc, 1), lambda: (0, 0)),
        ],
        out_specs=[
            pl.BlockSpec((1, 1), lambda: (0, 0)),
            pl.BlockSpec((1, 1), lambda: (0, 0)),
        ],
        out_shape=[
            jax.ShapeDtypeStruct((1, 1), jnp.float32),
            jax.ShapeDtypeStruct((1, 1), jnp.float32),
        ],
        interpret=_INTERPRET,
    )(dmin, se)

    pres = pl.pallas_call(
        functools.partial(_presence_body, nt=nt),
        grid=(_KP // _CC,),
        in_specs=[pl.BlockSpec((nloc, 1), lambda j: (0, 0))],
        out_specs=pl.BlockSpec((1, 1, _CC), lambda j: (j, 0, 0)),
        out_shape=jax.ShapeDtypeStruct((_KP // _CC, 1, _CC), jnp.float32),
        interpret=_INTERPRET,
    )(idx)

    if _INTERPRET:
        zq = jnp.take(enP, idx[:, 0], axis=0)
    else:
        ng = ((nloc + 127) // 128) * 128
        idxp = jnp.pad(idx.reshape(1, nloc), ((0, 0), (0, ng - nloc)))
        zq = _sc_gather(enP, idxp, ng)[:nloc]

    out2d = pl.pallas_call(
        _up_body,
        grid=(nt,),
        in_specs=[
            pl.BlockSpec((_TN, _MID), lambda t: (t, 0)),
            pl.BlockSpec((_MID, _C), lambda t: (0, 0)),
            pl.BlockSpec((1, _C), lambda t: (0, 0)),
        ],
        out_specs=pl.BlockSpec((_TN, _C), lambda t: (t, 0)),
        out_shape=jax.ShapeDtypeStruct((nloc, _C), jnp.float32),
        interpret=_INTERPRET,
    )(zq, W_up, b_up.reshape(1, _C))

    out = out2d.reshape(b_loc, _L, _C)

    vq = sdm[0, 0] / jnp.float32(_N * _MID)
    top1 = st1[0, 0] / jnp.float32(_N)
    usage = jnp.sum(pres.reshape(1, _KP)) / jnp.float32(_K)
    return out, vq, usage, top1


def kernel(x, calib, emb, W_down, b_down, W_up, b_up):
    del calib
    out, vq, usage, top1 = _shard_pipeline(
        x, emb, W_down, b_down, W_up, b_up)
    return (out, vq, 0.25 * vq, jnp.float32(0.0), usage, top1)


# speedup vs baseline: 1.4524x; 1.0010x over previous
import functools

import jax
import jax.numpy as jnp
from jax.experimental import pallas as pl
from jax.experimental.pallas import tpu as pltpu
from jax.experimental.pallas import tpu_sc as plsc

_B, _L, _C = 32, 450, 768
_MID = 256
_K = 8912
_KP = 9216
_N = _B * _L
_TN = 1800
_TK = 1024
_NK = _KP // _TK
_CC = 1024
_INV_TEMP = -1.0 / 0.07
_EPS = 1e-12

_PREC = jax.lax.Precision.DEFAULT


def _flash_body(zn_ref, en_ref, dmin_ref, idx_ref, se_ref,
                rmin_ref, ridx_ref, rsum_ref):
    k = pl.program_id(1)

    @pl.when(k == 0)
    def _():
        rmin_ref[...] = jnp.full((_TN, 1), 3e38, jnp.float32)
        ridx_ref[...] = jnp.full((_TN, 1), 2**30, jnp.int32)
        rsum_ref[...] = jnp.zeros((_TN, 1), jnp.float32)

    en_k = en_ref[:, pl.ds(k * _TK, _TK)]
    s = jnp.dot(zn_ref[...], en_k, precision=_PREC,
                preferred_element_type=jnp.float32)
    d = -2.0 * s + 2.0
    tmin = jnp.min(d, axis=1, keepdims=True)
    lane = jax.lax.broadcasted_iota(jnp.int32, (1, _TK), 1)
    tidx = jnp.min(jnp.where(d == tmin, lane, jnp.int32(2**30)),
                   axis=1, keepdims=True) + k * _TK
    e = jnp.exp(d * jnp.float32(_INV_TEMP))
    tsum = jnp.sum(e, axis=1, keepdims=True)

    better = tmin < rmin_ref[...]
    ridx_ref[...] = jnp.where(better, tidx, ridx_ref[...])
    rmin_ref[...] = jnp.where(better, tmin, rmin_ref[...])
    rsum_ref[...] = rsum_ref[...] + tsum

    @pl.when(k == _NK - 1)
    def _():
        dmin_ref[...] = rmin_ref[...]
        idx_ref[...] = ridx_ref[...]
        se_ref[...] = rsum_ref[...]


def _scalars_body(dmin_ref, se_ref, sdm_ref, st1_ref):
    dm = dmin_ref[...]
    padc = jnp.float32(_KP - _K) * jnp.exp(
        jnp.float32(2.0) * jnp.float32(_INV_TEMP))
    se = se_ref[...] - padc
    sdm_ref[...] = jnp.sum(dm).reshape(1, 1)
    t1 = jnp.exp(dm * jnp.float32(_INV_TEMP)) / se
    st1_ref[...] = jnp.sum(t1).reshape(1, 1)


def _presence_body(idx_ref, pres_ref, *, nt):
    j = pl.program_id(0)
    codes = jax.lax.broadcasted_iota(jnp.int32, (1, _CC), 1) + j * _CC

    def tok_tile(i, acc):
        chunk = idx_ref[pl.ds(i * _TN, _TN), :]
        eq = (chunk == codes).astype(jnp.float32)
        return jnp.maximum(acc, jnp.max(eq, axis=0, keepdims=True))

    pres = jax.lax.fori_loop(
        0, nt, tok_tile, jnp.zeros((1, _CC), jnp.float32))
    pres_ref[...] = pres.reshape(1, 1, _CC)


def _up_body(zq_ref, wup_ref, bup_ref, out_ref):
    out_ref[...] = jnp.dot(zq_ref[...], wup_ref[...], precision=_PREC,
                           preferred_element_type=jnp.float32) + bup_ref[...]


def _sc_gather(enP, idx2, ng):
    mesh = plsc.VectorSubcoreMesh(core_axis_name="core",
                                  subcore_axis_name="subcore")
    win = 128

    @functools.partial(
        pl.kernel,
        out_type=jax.ShapeDtypeStruct((ng, _MID), jnp.float32),
        mesh=mesh)
    def gk(en_hbm, i_hbm, o_hbm):
        def body(i_vmem, o_vmem):
            pltpu.sync_copy(en_hbm.at[i_vmem.at[0]], o_vmem)

        pltpu.emit_pipeline(
            body,
            grid=(ng // win,),
            in_specs=[pl.BlockSpec((1, win), lambda i: (0, i))],
            out_specs=[pl.BlockSpec((win, _MID), lambda i: (i, 0))],
            core_axis_name=("core", "subcore"),
            dimension_semantics=(pltpu.PARALLEL,),
        )(i_hbm, o_hbm)

    return gk(enP, idx2)


def _vq_pipeline(xs, emb, W_down, b_down, W_up, b_up):
    b_loc = xs.shape[0]
    nloc = b_loc * _L
    nt = nloc // _TN

    z = xs @ W_down + b_down
    zf = z.reshape(nloc, _MID)
    zn = zf / jnp.maximum(
        jnp.sqrt(jnp.sum(zf * zf, axis=-1, keepdims=True)), _EPS)
    en = emb / jnp.maximum(
        jnp.sqrt(jnp.sum(emb * emb, axis=-1, keepdims=True)), _EPS)
    enP = jnp.pad(en, ((0, _KP - _K), (0, 0)))
    enT = enP.T

    dmin, idx, se = pl.pallas_call(
        _flash_body,
        grid=(nt, _NK),
        in_specs=[
            pl.BlockSpec((_TN, _MID), lambda t, k: (t, 0)),
            pl.BlockSpec((_MID, _KP), lambda t, k: (0, 0)),
        ],
        out_specs=[
            pl.BlockSpec((_TN, 1), lambda t, k: (t, 0)),
            pl.BlockSpec((_TN, 1), lambda t, k: (t, 0)),
            pl.BlockSpec((_TN, 1), lambda t, k: (t, 0)),
        ],
        out_shape=[
            jax.ShapeDtypeStruct((nloc, 1), jnp.float32),
            jax.ShapeDtypeStruct((nloc, 1), jnp.int32),
            jax.ShapeDtypeStruct((nloc, 1), jnp.float32),
        ],
        scratch_shapes=[
            pltpu.VMEM((_TN, 1), jnp.float32),
            pltpu.VMEM((_TN, 1), jnp.int32),
            pltpu.VMEM((_TN, 1), jnp.float32),
        ],
    )(zn, enT)

    sdm, st1 = pl.pallas_call(
        _scalars_body,
        in_specs=[
            pl.BlockSpec((nloc, 1), lambda: (0, 0)),
            pl.BlockSpec((nloc, 1), lambda: (0, 0)),
        ],
        out_specs=[
            pl.BlockSpec((1, 1), lambda: (0, 0)),
            pl.BlockSpec((1, 1), lambda: (0, 0)),
        ],
        out_shape=[
            jax.ShapeDtypeStruct((1, 1), jnp.float32),
            jax.ShapeDtypeStruct((1, 1), jnp.float32),
        ],
    )(dmin, se)

    pres = pl.pallas_call(
        functools.partial(_presence_body, nt=nt),
        grid=(_KP // _CC,),
        in_specs=[pl.BlockSpec((nloc, 1), lambda j: (0, 0))],
        out_specs=pl.BlockSpec((1, 1, _CC), lambda j: (j, 0, 0)),
        out_shape=jax.ShapeDtypeStruct((_KP // _CC, 1, _CC), jnp.float32),
    )(idx)

    ng = ((nloc + 127) // 128) * 128
    idxp = jnp.pad(idx.reshape(1, nloc), ((0, 0), (0, ng - nloc)))
    zq = _sc_gather(enP, idxp, ng)[:nloc]

    out2d = pl.pallas_call(
        _up_body,
        grid=(nt,),
        in_specs=[
            pl.BlockSpec((_TN, _MID), lambda t: (t, 0)),
            pl.BlockSpec((_MID, _C), lambda t: (0, 0)),
            pl.BlockSpec((1, _C), lambda t: (0, 0)),
        ],
        out_specs=pl.BlockSpec((_TN, _C), lambda t: (t, 0)),
        out_shape=jax.ShapeDtypeStruct((nloc, _C), jnp.float32),
    )(zq, W_up, b_up.reshape(1, _C))

    out = out2d.reshape(b_loc, _L, _C)

    vq = sdm[0, 0] / jnp.float32(_N * _MID)
    top1 = st1[0, 0] / jnp.float32(_N)
    usage = jnp.sum(pres.reshape(1, _KP)) / jnp.float32(_K)
    return out, vq, usage, top1


def kernel(x, calib, emb, W_down, b_down, W_up, b_up):
    del calib
    out, vq, usage, top1 = _vq_pipeline(
        x, emb, W_down, b_down, W_up, b_up)
    return (out, vq, 0.25 * vq, jnp.float32(0.0), usage, top1)
